# Initial kernel scaffold; baseline (speedup 1.0000x reference)
import functools
import math

import jax
import jax.numpy as jnp
from jax import lax
from jax.experimental import pallas as pl
from jax.experimental.pallas import tpu as pltpu
from jax.experimental.pallas import tpu_sc as plsc

N = 10000
E = 320000
H = 64
D_EDGE = 16

NC = 2
NS = 16
NW = NC * NS
MB = 128
EPT = 10112
NMB = EPT // MB
EPAD = NW * EPT
N_AGG = 10112
ZPT = N_AGG // NS

_MESH = plsc.VectorSubcoreMesh(core_axis_name="c", subcore_axis_name="s")


@functools.partial(
    pl.kernel,
    mesh=_MESH,
    out_type=jax.ShapeDtypeStruct((EPAD, 2 * H), jnp.float32),
    scratch_types=[
        pltpu.VMEM((NMB, MB), jnp.int32),
        pltpu.VMEM((NMB, MB), jnp.int32),
        pltpu.VMEM((MB, 2 * H), jnp.float32),
        pltpu.VMEM((MB, 2 * H), jnp.float32),
        pltpu.SemaphoreType.DMA,
        pltpu.SemaphoreType.DMA,
    ],
)
def _sc_gather(psrc_hbm, pdst_hbm, src_hbm, dst_hbm, z_hbm,
               idxs_v, idxd_v, buf_a, buf_b, sem_a, sem_b):
    wid = lax.axis_index("s") * NC + lax.axis_index("c")
    row0 = wid * NMB
    base = wid * EPT
    pltpu.sync_copy(src_hbm.at[pl.ds(row0, NMB)], idxs_v)
    pltpu.sync_copy(dst_hbm.at[pl.ds(row0, NMB)], idxd_v)

    def body(j, carry):
        cp_a = pltpu.async_copy(psrc_hbm.at[idxs_v.at[j]], buf_a, sem_a)
        cp_b = pltpu.async_copy(pdst_hbm.at[idxd_v.at[j]], buf_b, sem_b)
        cp_a.wait()
        cp_b.wait()

        def add_row(r, c2):
            for k in range(8):
                sl = pl.ds(k * 16, 16)
                buf_a[r, sl] = buf_a[r, sl] + buf_b[r, sl]
            return c2

        lax.fori_loop(0, MB, add_row, 0)
        pltpu.sync_copy(buf_a, z_hbm.at[pl.ds(base + j * MB, MB)])
        return carry

    lax.fori_loop(0, NMB, body, 0)


@functools.partial(
    pl.kernel,
    mesh=_MESH,
    out_type=jax.ShapeDtypeStruct((NC, N, H), jnp.float32),
    scratch_types=[
        pltpu.VMEM_SHARED((N_AGG, H), jnp.float32),
        pltpu.VMEM((NMB, MB), jnp.int32),
        pltpu.VMEM((MB, H), jnp.float32),
    ],
)
def _sc_scatter(h_hbm, dst_hbm, agg_hbm, agg_sh, idx_v, rows_v):
    c = lax.axis_index("c")
    s = lax.axis_index("s")
    wid = c * NS + s
    row0 = wid * NMB
    base = wid * EPT

    def zrow(r, c2):
        for k in range(H // 16):
            rows_v[r, pl.ds(k * 16, 16)] = jnp.zeros((16,), jnp.float32)
        return c2

    lax.fori_loop(0, MB, zrow, 0)
    for q in range(4):
        pltpu.sync_copy(rows_v, agg_sh.at[pl.ds(s * ZPT + q * MB, MB)])
    pltpu.sync_copy(rows_v.at[pl.ds(0, ZPT - 4 * MB)],
                    agg_sh.at[pl.ds(s * ZPT + 4 * MB, ZPT - 4 * MB)])
    plsc.subcore_barrier()

    pltpu.sync_copy(dst_hbm.at[pl.ds(row0, NMB)], idx_v)

    def body(j, carry):
        pltpu.sync_copy(h_hbm.at[pl.ds(base + j * MB, MB)], rows_v)
        pltpu.sync_copy(rows_v, agg_sh.at[idx_v.at[j]], add=True)
        return carry

    lax.fori_loop(0, NMB, body, 0)
    plsc.subcore_barrier()

    @pl.when(s < 10)
    def _():
        pltpu.sync_copy(agg_sh.at[pl.ds(s * 1000, 1000)],
                        agg_hbm.at[c, pl.ds(s * 1000, 1000)])


_NBLK = 2000
_EBLK = 4096


def _t0_body(x_ref, we_ref, be_ref, ws_ref, wd_ref, v_ref, ps_ref, pd_ref):
    t = jnp.dot(x_ref[...], we_ref[...], preferred_element_type=jnp.float32)
    t = t + be_ref[...]
    v = t * jax.nn.sigmoid(t)
    v_ref[...] = v
    ps_ref[...] = jnp.dot(v, ws_ref[...], preferred_element_type=jnp.float32)
    pd_ref[...] = jnp.dot(v, wd_ref[...], preferred_element_type=jnp.float32)


def _t0(x, w_emb, b_emb, wsrc, wdst):
    g = N // _NBLK
    return pl.pallas_call(
        _t0_body,
        grid=(g,),
        in_specs=[
            pl.BlockSpec((_NBLK, 128), lambda i: (i, 0)),
            pl.BlockSpec((128, H), lambda i: (0, 0)),
            pl.BlockSpec((1, H), lambda i: (0, 0)),
            pl.BlockSpec((H, 2 * H), lambda i: (0, 0)),
            pl.BlockSpec((H, 2 * H), lambda i: (0, 0)),
        ],
        out_specs=[
            pl.BlockSpec((_NBLK, H), lambda i: (i, 0)),
            pl.BlockSpec((_NBLK, 2 * H), lambda i: (i, 0)),
            pl.BlockSpec((_NBLK, 2 * H), lambda i: (i, 0)),
        ],
        out_shape=[
            jax.ShapeDtypeStruct((N, H), jnp.float32),
            jax.ShapeDtypeStruct((N, 2 * H), jnp.float32),
            jax.ShapeDtypeStruct((N, 2 * H), jnp.float32),
        ],
    )(x, w_emb, b_emb, wsrc, wdst)


def _t1_body(z_ref, ef_ref, we_ref, b_ref, h_ref):
    i = pl.program_id(0)
    z = z_ref[...] + jnp.dot(ef_ref[...], we_ref[...],
                             preferred_element_type=jnp.float32) + b_ref[...]
    am = jax.nn.sigmoid(z[:, :H])
    sp = jax.nn.softplus(z[:, H:])
    h = am * sp
    rows = i * _EBLK + lax.broadcasted_iota(jnp.int32, (_EBLK, 1), 0)
    h_ref[...] = jnp.where(rows < E, h, 0.0)


def _t1(z, ef, we, b):
    g = EPAD // _EBLK
    return pl.pallas_call(
        _t1_body,
        grid=(g,),
        in_specs=[
            pl.BlockSpec((_EBLK, 2 * H), lambda i: (i, 0)),
            pl.BlockSpec((_EBLK, D_EDGE), lambda i: (i, 0)),
            pl.BlockSpec((D_EDGE, 2 * H), lambda i: (0, 0)),
            pl.BlockSpec((1, 2 * H), lambda i: (0, 0)),
        ],
        out_specs=pl.BlockSpec((_EBLK, H), lambda i: (i, 0)),
        out_shape=jax.ShapeDtypeStruct((EPAD, H), jnp.float32),
    )(z, ef, we, b)


def _t2_body(agg_ref, v_ref, cg_ref, be_ref, ws_ref, wd_ref,
             vn_ref, ps_ref, pd_ref):
    a = agg_ref[0] + agg_ref[1]
    vn = jax.nn.softplus(a * cg_ref[...] + be_ref[...] + v_ref[...])
    vn_ref[...] = vn
    ps_ref[...] = jnp.dot(vn, ws_ref[...], preferred_element_type=jnp.float32)
    pd_ref[...] = jnp.dot(vn, wd_ref[...], preferred_element_type=jnp.float32)


def _t2(agg, v, cg, be, wsrc, wdst):
    g = N // _NBLK
    return pl.pallas_call(
        _t2_body,
        grid=(g,),
        in_specs=[
            pl.BlockSpec((NC, _NBLK, H), lambda i: (0, i, 0)),
            pl.BlockSpec((_NBLK, H), lambda i: (i, 0)),
            pl.BlockSpec((1, H), lambda i: (0, 0)),
            pl.BlockSpec((1, H), lambda i: (0, 0)),
            pl.BlockSpec((H, 2 * H), lambda i: (0, 0)),
            pl.BlockSpec((H, 2 * H), lambda i: (0, 0)),
        ],
        out_specs=[
            pl.BlockSpec((_NBLK, H), lambda i: (i, 0)),
            pl.BlockSpec((_NBLK, 2 * H), lambda i: (i, 0)),
            pl.BlockSpec((_NBLK, 2 * H), lambda i: (i, 0)),
        ],
        out_shape=[
            jax.ShapeDtypeStruct((N, H), jnp.float32),
            jax.ShapeDtypeStruct((N, 2 * H), jnp.float32),
            jax.ShapeDtypeStruct((N, 2 * H), jnp.float32),
        ],
    )(agg, v, cg, be, wsrc, wdst)


def _t2f_body(agg_ref, v_ref, cg_ref, be_ref, acc_ref):
    i = pl.program_id(0)
    a = agg_ref[0] + agg_ref[1]
    vn = jax.nn.softplus(a * cg_ref[...] + be_ref[...] + v_ref[...])
    part = jnp.sum(vn, axis=0, keepdims=True)

    @pl.when(i == 0)
    def _():
        acc_ref[...] = jnp.zeros_like(acc_ref)

    acc_ref[...] += part


def _t2f(agg, v, cg, be):
    g = N // _NBLK
    return pl.pallas_call(
        _t2f_body,
        grid=(g,),
        in_specs=[
            pl.BlockSpec((NC, _NBLK, H), lambda i: (0, i, 0)),
            pl.BlockSpec((_NBLK, H), lambda i: (i, 0)),
            pl.BlockSpec((1, H), lambda i: (0, 0)),
            pl.BlockSpec((1, H), lambda i: (0, 0)),
        ],
        out_specs=pl.BlockSpec((1, H), lambda i: (0, 0)),
        out_shape=jax.ShapeDtypeStruct((1, H), jnp.float32),
    )(agg, v, cg, be)


def _t3_body(vs_ref, wfc_ref, bfc_ref, wp_ref, bp_ref, out_ref):
    vc = vs_ref[...] * (1.0 / N)
    t = jnp.dot(vc, wfc_ref[...], preferred_element_type=jnp.float32)
    t = t + bfc_ref[...]
    t = t * jax.nn.sigmoid(t)
    out_ref[...] = jnp.dot(t, wp_ref[...],
                           preferred_element_type=jnp.float32) + bp_ref[...]


def _t3(vsum, wfc, bfc, wp, bp):
    return pl.pallas_call(
        _t3_body,
        out_shape=jax.ShapeDtypeStruct((1, 1), jnp.float32),
    )(vsum, wfc, bfc, wp, bp)


def kernel(node_feats, edge_feats, edge_index, W_emb, b_emb, g_emb, be_emb,
           conv_mlp_W, conv_mlp_b, conv_mlp_g, conv_mlp_be,
           conv_scr_W, conv_scr_b, conv_scr_g, conv_scr_be,
           conv_bn_g, conv_bn_be, W_fc, b_fc, g_fc, be_fc, W_pred, b_pred):
    c = 1.0 / math.sqrt(1.0 + 1e-5)

    w_embf = W_emb * (c * g_emb)[None, :]
    b_embf = (b_emb * c * g_emb + be_emb)[None, :]
    wm = conv_mlp_W * (c * conv_mlp_g)[:, None, :]
    bm = conv_mlp_b * (c * conv_mlp_g) + conv_mlp_be
    ws = conv_scr_W * (c * conv_scr_g)[:, None, :]
    bs = conv_scr_b * (c * conv_scr_g) + conv_scr_be
    wsrc = jnp.concatenate([wm[:, :H, :], ws[:, :H, :]], axis=2)
    wdst = jnp.concatenate([wm[:, H:2 * H, :], ws[:, H:2 * H, :]], axis=2)
    wed = jnp.concatenate([wm[:, 2 * H:, :], ws[:, 2 * H:, :]], axis=2)
    bcat = jnp.concatenate([bm, bs], axis=1)[:, None, :]
    cg_bn = (c * conv_bn_g)[:, None, :]
    be_bn = conv_bn_be[:, None, :]
    w_fcf = W_fc * (c * g_fc)[None, :]
    b_fcf = (b_fc * c * g_fc + be_fc)[None, :]
    b_predf = b_pred[None, :]

    src = edge_index[0].astype(jnp.int32)
    dst = edge_index[1].astype(jnp.int32)
    src2d = jnp.pad(src, (0, EPAD - E)).reshape(NW * NMB, MB)
    dst2d = jnp.pad(dst, (0, EPAD - E)).reshape(NW * NMB, MB)
    efp = jnp.pad(edge_feats, ((0, EPAD - E), (0, 0)))

    v, psrc, pdst = _t0(node_feats, w_embf, b_embf, wsrc[0], wdst[0])
    for i in range(3):
        z = _sc_gather(psrc, pdst, src2d, dst2d)
        h = _t1(z, efp, wed[i], bcat[i])
        agg = _sc_scatter(h, dst2d)
        if i < 2:
            v, psrc, pdst = _t2(agg, v, cg_bn[i], be_bn[i],
                                wsrc[i + 1], wdst[i + 1])
        else:
            vsum = _t2f(agg, v, cg_bn[i], be_bn[i])
    return _t3(vsum, w_fcf, b_fcf, W_pred, b_predf)


# speedup vs baseline: 2.3969x; 2.3969x over previous
import functools
import math

import jax
import jax.numpy as jnp
from jax import lax
from jax.experimental import pallas as pl
from jax.experimental.pallas import tpu as pltpu
from jax.experimental.pallas import tpu_sc as plsc

N = 10000
E = 320000
H = 64
D_EDGE = 16

NC = 2
NS = 16
NW = NC * NS
MB = 128
EPT = 10112
NMB = EPT // MB
EPAD = NW * EPT
N_AGG = 10112
ZPT = N_AGG // NS

def _sc_gather_body(psrc_hbm, pdst_hbm, src_hbm, dst_hbm, z_hbm,
                    idxs_v, idxd_v, buf_a, buf_b, sem_a, sem_b):
    wid = lax.axis_index("s") * NC + lax.axis_index("c")
    base = wid * EPT
    pltpu.sync_copy(src_hbm.at[wid], idxs_v)
    pltpu.sync_copy(dst_hbm.at[wid], idxd_v)

    def body(j, carry):
        cp_a = pltpu.async_copy(psrc_hbm.at[idxs_v.at[j]], buf_a, sem_a)
        cp_b = pltpu.async_copy(pdst_hbm.at[idxd_v.at[j]], buf_b, sem_b)
        cp_a.wait()
        cp_b.wait()

        def add_row(r, c2):
            for k in range(8):
                sl = pl.ds(k * 16, 16)
                buf_a[r, sl] = buf_a[r, sl] + buf_b[r, sl]
            return c2

        lax.fori_loop(0, MB, add_row, 0)
        pltpu.sync_copy(buf_a, z_hbm.at[pl.ds(base + j * MB, MB)])
        return carry

    lax.fori_loop(0, NMB, body, 0)


@functools.cache
def _sc_gather():
    mesh = plsc.VectorSubcoreMesh(core_axis_name="c", subcore_axis_name="s")
    return pl.kernel(
        _sc_gather_body,
        mesh=mesh,
        out_type=jax.ShapeDtypeStruct((EPAD, 2 * H), jnp.float32),
        scratch_types=[
            pltpu.VMEM((NMB + 1, MB), jnp.int32),
            pltpu.VMEM((NMB + 1, MB), jnp.int32),
            pltpu.VMEM((MB, 2 * H), jnp.float32),
            pltpu.VMEM((MB, 2 * H), jnp.float32),
            pltpu.SemaphoreType.DMA,
            pltpu.SemaphoreType.DMA,
        ],
    )


def _sc_scatter_body(h_hbm, dst_hbm, agg_hbm, agg_sh, idx_v, rows_v):
    c = lax.axis_index("c")
    s = lax.axis_index("s")
    wid = c * NS + s
    base = wid * EPT

    def zrow(r, c2):
        for k in range(2 * H // 16):
            rows_v[r, pl.ds(k * 16, 16)] = jnp.zeros((16,), jnp.float32)
        return c2

    lax.fori_loop(0, MB, zrow, 0)
    for q in range(4):
        pltpu.sync_copy(rows_v, agg_sh.at[pl.ds(s * ZPT + q * MB, MB)])
    pltpu.sync_copy(rows_v.at[pl.ds(0, ZPT - 4 * MB)],
                    agg_sh.at[pl.ds(s * ZPT + 4 * MB, ZPT - 4 * MB)])
    plsc.subcore_barrier()

    pltpu.sync_copy(dst_hbm.at[wid], idx_v)

    def body(j, carry):
        pltpu.sync_copy(h_hbm.at[pl.ds(base + j * MB, MB)], rows_v)
        pltpu.sync_copy(rows_v, agg_sh.at[idx_v.at[j]], add=True)
        return carry

    lax.fori_loop(0, NMB, body, 0)
    plsc.subcore_barrier()

    @pl.when(s < 10)
    def _():
        pltpu.sync_copy(agg_sh.at[pl.ds(s * 1000, 1000)],
                        agg_hbm.at[c, pl.ds(s * 1000, 1000)])


@functools.cache
def _sc_scatter():
    mesh = plsc.VectorSubcoreMesh(core_axis_name="c", subcore_axis_name="s")
    return pl.kernel(
        _sc_scatter_body,
        mesh=mesh,
        out_type=jax.ShapeDtypeStruct((NC, N, 2 * H), jnp.float32),
        scratch_types=[
            pltpu.VMEM_SHARED((N_AGG, 2 * H), jnp.float32),
            pltpu.VMEM((NMB + 1, MB), jnp.int32),
            pltpu.VMEM((MB, 2 * H), jnp.float32),
        ],
    )


_NBLK = 2000
_EBLK = 4096


def _t0_body(x_ref, we_ref, be_ref, ws_ref, wd_ref, v_ref, ps_ref, pd_ref):
    t = jnp.dot(x_ref[...], we_ref[...], preferred_element_type=jnp.float32)
    t = t + be_ref[...]
    v = t * jax.nn.sigmoid(t)
    v_ref[...] = v
    ps_ref[...] = jnp.dot(v, ws_ref[...], preferred_element_type=jnp.float32)
    pd_ref[...] = jnp.dot(v, wd_ref[...], preferred_element_type=jnp.float32)


def _t0(x, w_emb, b_emb, wsrc, wdst):
    g = N // _NBLK
    return pl.pallas_call(
        _t0_body,
        grid=(g,),
        in_specs=[
            pl.BlockSpec((_NBLK, 128), lambda i: (i, 0)),
            pl.BlockSpec((128, H), lambda i: (0, 0)),
            pl.BlockSpec((1, H), lambda i: (0, 0)),
            pl.BlockSpec((H, 2 * H), lambda i: (0, 0)),
            pl.BlockSpec((H, 2 * H), lambda i: (0, 0)),
        ],
        out_specs=[
            pl.BlockSpec((_NBLK, H), lambda i: (i, 0)),
            pl.BlockSpec((_NBLK, 2 * H), lambda i: (i, 0)),
            pl.BlockSpec((_NBLK, 2 * H), lambda i: (i, 0)),
        ],
        out_shape=[
            jax.ShapeDtypeStruct((N, H), jnp.float32),
            jax.ShapeDtypeStruct((N, 2 * H), jnp.float32),
            jax.ShapeDtypeStruct((N, 2 * H), jnp.float32),
        ],
    )(x, w_emb, b_emb, wsrc, wdst)


def _t1_body(z_ref, ef_ref, we_ref, b_ref, h_ref):
    i = pl.program_id(0)
    z = z_ref[...] + jnp.dot(ef_ref[...], we_ref[...],
                             preferred_element_type=jnp.float32) + b_ref[...]
    am = jax.nn.sigmoid(z[:, :H])
    sp = jax.nn.softplus(z[:, H:])
    h = am * sp
    rows = i * _EBLK + lax.broadcasted_iota(jnp.int32, (_EBLK, 1), 0)
    h = jnp.where(rows < E, h, 0.0)
    h_ref[...] = jnp.concatenate([h, jnp.zeros_like(h)], axis=1)


def _t1(z, ef, we, b):
    g = EPAD // _EBLK
    return pl.pallas_call(
        _t1_body,
        grid=(g,),
        in_specs=[
            pl.BlockSpec((_EBLK, 2 * H), lambda i: (i, 0)),
            pl.BlockSpec((_EBLK, D_EDGE), lambda i: (i, 0)),
            pl.BlockSpec((D_EDGE, 2 * H), lambda i: (0, 0)),
            pl.BlockSpec((1, 2 * H), lambda i: (0, 0)),
        ],
        out_specs=pl.BlockSpec((_EBLK, 2 * H), lambda i: (i, 0)),
        out_shape=jax.ShapeDtypeStruct((EPAD, 2 * H), jnp.float32),
    )(z, ef, we, b)


def _t2_body(agg_ref, v_ref, cg_ref, be_ref, ws_ref, wd_ref,
             vn_ref, ps_ref, pd_ref):
    a = agg_ref[0, :, :H] + agg_ref[1, :, :H]
    vn = jax.nn.softplus(a * cg_ref[...] + be_ref[...] + v_ref[...])
    vn_ref[...] = vn
    ps_ref[...] = jnp.dot(vn, ws_ref[...], preferred_element_type=jnp.float32)
    pd_ref[...] = jnp.dot(vn, wd_ref[...], preferred_element_type=jnp.float32)


def _t2(agg, v, cg, be, wsrc, wdst):
    g = N // _NBLK
    return pl.pallas_call(
        _t2_body,
        grid=(g,),
        in_specs=[
            pl.BlockSpec((NC, _NBLK, 2 * H), lambda i: (0, i, 0)),
            pl.BlockSpec((_NBLK, H), lambda i: (i, 0)),
            pl.BlockSpec((1, H), lambda i: (0, 0)),
            pl.BlockSpec((1, H), lambda i: (0, 0)),
            pl.BlockSpec((H, 2 * H), lambda i: (0, 0)),
            pl.BlockSpec((H, 2 * H), lambda i: (0, 0)),
        ],
        out_specs=[
            pl.BlockSpec((_NBLK, H), lambda i: (i, 0)),
            pl.BlockSpec((_NBLK, 2 * H), lambda i: (i, 0)),
            pl.BlockSpec((_NBLK, 2 * H), lambda i: (i, 0)),
        ],
        out_shape=[
            jax.ShapeDtypeStruct((N, H), jnp.float32),
            jax.ShapeDtypeStruct((N, 2 * H), jnp.float32),
            jax.ShapeDtypeStruct((N, 2 * H), jnp.float32),
        ],
    )(agg, v, cg, be, wsrc, wdst)


def _t2f_body(agg_ref, v_ref, cg_ref, be_ref, acc_ref):
    i = pl.program_id(0)
    a = agg_ref[0, :, :H] + agg_ref[1, :, :H]
    vn = jax.nn.softplus(a * cg_ref[...] + be_ref[...] + v_ref[...])
    part = jnp.sum(vn, axis=0, keepdims=True)

    @pl.when(i == 0)
    def _():
        acc_ref[...] = jnp.zeros_like(acc_ref)

    acc_ref[...] += part


def _t2f(agg, v, cg, be):
    g = N // _NBLK
    return pl.pallas_call(
        _t2f_body,
        grid=(g,),
        in_specs=[
            pl.BlockSpec((NC, _NBLK, 2 * H), lambda i: (0, i, 0)),
            pl.BlockSpec((_NBLK, H), lambda i: (i, 0)),
            pl.BlockSpec((1, H), lambda i: (0, 0)),
            pl.BlockSpec((1, H), lambda i: (0, 0)),
        ],
        out_specs=pl.BlockSpec((1, H), lambda i: (0, 0)),
        out_shape=jax.ShapeDtypeStruct((1, H), jnp.float32),
    )(agg, v, cg, be)


def _t3_body(vs_ref, wfc_ref, bfc_ref, wp_ref, bp_ref, out_ref):
    vc = vs_ref[...] * (1.0 / N)
    t = jnp.dot(vc, wfc_ref[...], preferred_element_type=jnp.float32)
    t = t + bfc_ref[...]
    t = t * jax.nn.sigmoid(t)
    out_ref[...] = jnp.dot(t, wp_ref[...],
                           preferred_element_type=jnp.float32) + bp_ref[...]


def _t3(vsum, wfc, bfc, wp, bp):
    return pl.pallas_call(
        _t3_body,
        out_shape=jax.ShapeDtypeStruct((1, 1), jnp.float32),
    )(vsum, wfc, bfc, wp, bp)


def kernel(node_feats, edge_feats, edge_index, W_emb, b_emb, g_emb, be_emb,
           conv_mlp_W, conv_mlp_b, conv_mlp_g, conv_mlp_be,
           conv_scr_W, conv_scr_b, conv_scr_g, conv_scr_be,
           conv_bn_g, conv_bn_be, W_fc, b_fc, g_fc, be_fc, W_pred, b_pred):
    c = 1.0 / math.sqrt(1.0 + 1e-5)

    w_embf = W_emb * (c * g_emb)[None, :]
    b_embf = (b_emb * c * g_emb + be_emb)[None, :]
    wm = conv_mlp_W * (c * conv_mlp_g)[:, None, :]
    bm = conv_mlp_b * (c * conv_mlp_g) + conv_mlp_be
    ws = conv_scr_W * (c * conv_scr_g)[:, None, :]
    bs = conv_scr_b * (c * conv_scr_g) + conv_scr_be
    wsrc = jnp.concatenate([wm[:, :H, :], ws[:, :H, :]], axis=2)
    wdst = jnp.concatenate([wm[:, H:2 * H, :], ws[:, H:2 * H, :]], axis=2)
    wed = jnp.concatenate([wm[:, 2 * H:, :], ws[:, 2 * H:, :]], axis=2)
    bcat = jnp.concatenate([bm, bs], axis=1)[:, None, :]
    cg_bn = (c * conv_bn_g)[:, None, :]
    be_bn = conv_bn_be[:, None, :]
    w_fcf = W_fc * (c * g_fc)[None, :]
    b_fcf = (b_fc * c * g_fc + be_fc)[None, :]
    b_predf = b_pred[None, :]

    src = edge_index[0].astype(jnp.int32)
    dst = edge_index[1].astype(jnp.int32)
    src2d = jnp.pad(jnp.pad(src, (0, EPAD - E)).reshape(NW, NMB, MB),
                    ((0, 0), (0, 1), (0, 0)))
    dst2d = jnp.pad(jnp.pad(dst, (0, EPAD - E)).reshape(NW, NMB, MB),
                    ((0, 0), (0, 1), (0, 0)))
    efp = jnp.pad(edge_feats, ((0, EPAD - E), (0, 0)))

    v, psrc, pdst = _t0(node_feats, w_embf, b_embf, wsrc[0], wdst[0])
    for i in range(3):
        z = _sc_gather()(psrc, pdst, src2d, dst2d)
        h = _t1(z, efp, wed[i], bcat[i])
        agg = _sc_scatter()(h, dst2d)
        if i < 2:
            v, psrc, pdst = _t2(agg, v, cg_bn[i], be_bn[i],
                                wsrc[i + 1], wdst[i + 1])
        else:
            vsum = _t2f(agg, v, cg_bn[i], be_bn[i])
    return _t3(vsum, w_fcf, b_fcf, W_pred, b_predf)
